# Initial kernel scaffold; baseline (speedup 1.0000x reference)
import jax
import jax.numpy as jnp
from jax.experimental import pallas as pl


def kernel(x, edge_index, W1, attn_l1, attn_r1, b1, W2, attn_l2, attn_r2, b2):
    raise NotImplementedError("write your pallas kernel here")


# speedup vs baseline: 54.9944x; 54.9944x over previous
import functools

import jax
import jax.numpy as jnp
from jax import lax
from jax.experimental import pallas as pl
from jax.experimental.pallas import tpu as pltpu
from jax.experimental.pallas import tpu_sc as plsc

N = 10000
E = 320000
IN = 128
HD1 = 64
W1ROW = 80
HD2 = 16
W2ROW = 32

NW = 32
PT = E // NW
CH = 80
NCH = PT // CH
NP = 10240
RPT = NP // 16
ZB = 128

_BLK = 1000
_GRID = N // _BLK


def _tc_pre1(x_ref, w1_ref, al_ref, ar_ref, t1_ref, er_ref):
    feat = jnp.dot(x_ref[...], w1_ref[...],
                   preferred_element_type=jnp.float32,
                   precision=lax.Precision.HIGHEST)
    el = jnp.dot(feat, al_ref[...], preferred_element_type=jnp.float32,
                 precision=lax.Precision.HIGHEST)
    er = jnp.dot(feat, ar_ref[...], preferred_element_type=jnp.float32,
                 precision=lax.Precision.HIGHEST)
    z8 = jnp.zeros((_BLK, 8), jnp.float32)
    t1_ref[...] = jnp.concatenate([feat, el, z8], axis=1)
    er_ref[...] = jnp.concatenate([er, z8], axis=1)


def _tc_mid(p_ref, b1_ref, em_ref, w2_ref, al2_ref, ar2_ref, t2_ref, er2_ref):
    a = p_ref[0] + p_ref[1]
    msg = a[:, 0:HD1]
    den = jnp.dot(a[:, HD1:HD1 + 8], em_ref[...],
                  preferred_element_type=jnp.float32,
                  precision=lax.Precision.HIGHEST) + 1e-9
    h = msg / den + b1_ref[...]
    h = jnp.where(h > 0, h, jnp.exp(jnp.minimum(h, 0.0)) - 1.0)
    f2 = jnp.dot(h, w2_ref[...], preferred_element_type=jnp.float32,
                 precision=lax.Precision.HIGHEST)
    el2 = jnp.dot(f2, al2_ref[...], preferred_element_type=jnp.float32,
                  precision=lax.Precision.HIGHEST)
    er2 = jnp.dot(f2, ar2_ref[...], preferred_element_type=jnp.float32,
                  precision=lax.Precision.HIGHEST)
    ones = jnp.ones((1, HD2), jnp.float32)
    t2_ref[...] = jnp.concatenate([f2, el2 * ones], axis=1)
    er2_ref[...] = er2 * ones


def _tc_post(p_ref, b2_ref, o_ref):
    a = p_ref[0] + p_ref[1]
    o_ref[...] = a[:, 0:HD2] / (a[:, HD2:HD2 + 1] + 1e-9) + b2_ref[...]


def _sc_edges1(tab_hbm, er_hbm, src_hbm, dst_hbm, out_hbm,
               src_v, dst_v, rows, erc, msg, zbuf, acc, sem):
    c = lax.axis_index("c")
    s = lax.axis_index("s")
    wid = s * 2 + c

    def _zrow(i, _):
        zv = jnp.zeros((16,), jnp.float32)
        for q in range(W1ROW // 16):
            zbuf[i, pl.ds(q * 16, 16)] = zv
        return 0
    lax.fori_loop(0, ZB, _zrow, 0)
    for k in range(RPT // ZB):
        pltpu.sync_copy(zbuf, acc.at[pl.ds(s * RPT + k * ZB, ZB)])
    plsc.subcore_barrier()

    pltpu.sync_copy(src_hbm.at[wid], src_v)
    pltpu.sync_copy(dst_hbm.at[wid], dst_v)

    lane = lax.iota(jnp.int32, 16)
    half = lane >> 3

    def _chunk(j, _):
        pltpu.async_copy(tab_hbm.at[src_v.at[j]], rows, sem).wait()
        pltpu.async_copy(er_hbm.at[dst_v.at[j]], erc, sem).wait()

        def _edge(e, _):
            elv = rows[e, pl.ds(HD1, 16)]
            erv = erc[e, pl.ds(0, 16)]
            t = elv + erv
            t = jnp.maximum(t, 0.2 * t)
            ee = jnp.exp(t)
            for q in range(4):
                mq = lax.gather(
                    ee, (half + 2 * q)[:, None],
                    lax.GatherDimensionNumbers(
                        offset_dims=(), collapsed_slice_dims=(0,),
                        start_index_map=(0,)),
                    (1,), mode=lax.GatherScatterMode.PROMISE_IN_BOUNDS)
                msg[e, pl.ds(q * 16, 16)] = rows[e, pl.ds(q * 16, 16)] * mq
            msg[e, pl.ds(HD1, 16)] = jnp.where(lane < 8, ee, 0.0)
            return 0
        lax.fori_loop(0, CH, _edge, 0)
        pltpu.sync_copy(msg, acc.at[dst_v.at[j]], add=True)
        return 0
    lax.fori_loop(0, NCH, _chunk, 0)

    plsc.subcore_barrier()
    pltpu.sync_copy(acc.at[pl.ds(s * RPT, RPT)],
                    out_hbm.at[c, pl.ds(s * RPT, RPT)])


def _sc_edges2(tab_hbm, er_hbm, src_hbm, dst_hbm, out_hbm,
               src_v, dst_v, rows, erc, msg, zbuf, acc, sem):
    c = lax.axis_index("c")
    s = lax.axis_index("s")
    wid = s * 2 + c

    def _zrow(i, _):
        zv = jnp.zeros((16,), jnp.float32)
        for q in range(W2ROW // 16):
            zbuf[i, pl.ds(q * 16, 16)] = zv
        return 0
    lax.fori_loop(0, ZB, _zrow, 0)
    for k in range(RPT // ZB):
        pltpu.sync_copy(zbuf, acc.at[pl.ds(s * RPT + k * ZB, ZB)])
    plsc.subcore_barrier()

    pltpu.sync_copy(src_hbm.at[wid], src_v)
    pltpu.sync_copy(dst_hbm.at[wid], dst_v)

    lane = lax.iota(jnp.int32, 16)

    def _chunk(j, _):
        pltpu.async_copy(tab_hbm.at[src_v.at[j]], rows, sem).wait()
        pltpu.async_copy(er_hbm.at[dst_v.at[j]], erc, sem).wait()

        def _edge(e, _):
            t = rows[e, pl.ds(HD2, 16)] + erc[e, pl.ds(0, 16)]
            t = jnp.maximum(t, 0.2 * t)
            ee = jnp.exp(t)
            msg[e, pl.ds(0, 16)] = rows[e, pl.ds(0, 16)] * ee
            msg[e, pl.ds(HD2, 16)] = jnp.where(lane < 1, ee, 0.0)
            return 0
        lax.fori_loop(0, CH, _edge, 0)
        pltpu.sync_copy(msg, acc.at[dst_v.at[j]], add=True)
        return 0
    lax.fori_loop(0, NCH, _chunk, 0)

    plsc.subcore_barrier()
    pltpu.sync_copy(acc.at[pl.ds(s * RPT, RPT)],
                    out_hbm.at[c, pl.ds(s * RPT, RPT)])


def _sc_call(body, row_w):
    mesh = plsc.VectorSubcoreMesh(core_axis_name="c", subcore_axis_name="s")
    return pl.kernel(
        body,
        out_type=jax.ShapeDtypeStruct((2, NP, row_w), jnp.float32),
        mesh=mesh,
        scratch_types=[
            pltpu.VMEM((NCH, CH), jnp.int32),
            pltpu.VMEM((NCH, CH), jnp.int32),
            pltpu.VMEM((CH, row_w), jnp.float32),
            pltpu.VMEM((CH, 16), jnp.float32),
            pltpu.VMEM((CH, row_w), jnp.float32),
            pltpu.VMEM((ZB, row_w), jnp.float32),
            pltpu.VMEM_SHARED((NP, row_w), jnp.float32),
            pltpu.SemaphoreType.DMA,
        ],
        compiler_params=pltpu.CompilerParams(use_tc_tiling_on_sc=False),
    )


def kernel(x, edge_index, W1, attn_l1, attn_r1, b1, W2, attn_l2, attn_r2, b2):
    f32 = jnp.float32
    src = edge_index[0].astype(jnp.int32).reshape(NW, NCH, CH)
    dst = edge_index[1].astype(jnp.int32).reshape(NW, NCH, CH)

    eye8 = jnp.eye(8, dtype=f32)
    AL = (eye8[:, None, :] * attn_l1.astype(f32)[:, :, None]).reshape(HD1, 8)
    AR = (eye8[:, None, :] * attn_r1.astype(f32)[:, :, None]).reshape(HD1, 8)
    Emat = jnp.repeat(eye8, 8, axis=1)

    t1, er1 = pl.pallas_call(
        _tc_pre1,
        grid=(_GRID,),
        in_specs=[
            pl.BlockSpec((_BLK, IN), lambda i: (i, 0)),
            pl.BlockSpec((IN, HD1), lambda i: (0, 0)),
            pl.BlockSpec((HD1, 8), lambda i: (0, 0)),
            pl.BlockSpec((HD1, 8), lambda i: (0, 0)),
        ],
        out_specs=[
            pl.BlockSpec((_BLK, W1ROW), lambda i: (i, 0)),
            pl.BlockSpec((_BLK, 16), lambda i: (i, 0)),
        ],
        out_shape=[
            jax.ShapeDtypeStruct((N, W1ROW), f32),
            jax.ShapeDtypeStruct((N, 16), f32),
        ],
    )(x.astype(f32), W1.astype(f32), AL, AR)

    parts1 = _sc_call(_sc_edges1, W1ROW)(t1, er1, src, dst)

    t2, er2 = pl.pallas_call(
        _tc_mid,
        grid=(_GRID,),
        in_specs=[
            pl.BlockSpec((2, _BLK, W1ROW), lambda i: (0, i, 0)),
            pl.BlockSpec((1, HD1), lambda i: (0, 0)),
            pl.BlockSpec((8, HD1), lambda i: (0, 0)),
            pl.BlockSpec((HD1, HD2), lambda i: (0, 0)),
            pl.BlockSpec((HD2, 1), lambda i: (0, 0)),
            pl.BlockSpec((HD2, 1), lambda i: (0, 0)),
        ],
        out_specs=[
            pl.BlockSpec((_BLK, W2ROW), lambda i: (i, 0)),
            pl.BlockSpec((_BLK, 16), lambda i: (i, 0)),
        ],
        out_shape=[
            jax.ShapeDtypeStruct((N, W2ROW), f32),
            jax.ShapeDtypeStruct((N, 16), f32),
        ],
    )(parts1, b1.astype(f32).reshape(1, HD1), Emat, W2.astype(f32),
      attn_l2.astype(f32).reshape(HD2, 1), attn_r2.astype(f32).reshape(HD2, 1))

    parts2 = _sc_call(_sc_edges2, W2ROW)(t2, er2, src, dst)

    out = pl.pallas_call(
        _tc_post,
        grid=(_GRID,),
        in_specs=[
            pl.BlockSpec((2, _BLK, W2ROW), lambda i: (0, i, 0)),
            pl.BlockSpec((1, HD2), lambda i: (0, 0)),
        ],
        out_specs=pl.BlockSpec((_BLK, HD2), lambda i: (i, 0)),
        out_shape=jax.ShapeDtypeStruct((N, HD2), f32),
    )(parts2, b2.astype(f32).reshape(1, HD2))

    return out


# speedup vs baseline: 99.0423x; 1.8010x over previous
import functools

import jax
import jax.numpy as jnp
from jax import lax
from jax.experimental import pallas as pl
from jax.experimental.pallas import tpu as pltpu
from jax.experimental.pallas import tpu_sc as plsc

N = 10000
E = 320000
IN = 128
HD1 = 64
W1ROW = 80
HD2 = 16
W2ROW = 32

NW = 32
PT = E // NW
CH = 100
NCH = PT // CH
NP = 10240
RPT = NP // 16
ZB = 128

_BLK = 1000
_GRID = N // _BLK


def _tc_pre1(x_ref, w1_ref, al_ref, ar_ref, t1_ref, er_ref):
    feat = jnp.dot(x_ref[...], w1_ref[...],
                   preferred_element_type=jnp.float32,
                   precision=lax.Precision.HIGHEST)
    el = jnp.dot(feat, al_ref[...], preferred_element_type=jnp.float32,
                 precision=lax.Precision.HIGHEST)
    er = jnp.dot(feat, ar_ref[...], preferred_element_type=jnp.float32,
                 precision=lax.Precision.HIGHEST)
    z8 = jnp.zeros((_BLK, 8), jnp.float32)
    t1_ref[...] = jnp.concatenate([feat, el, z8], axis=1)
    er_ref[...] = jnp.concatenate([er, z8], axis=1)


def _tc_mid(p_ref, b1_ref, em_ref, w2_ref, al2_ref, ar2_ref, t2_ref, er2_ref):
    a = p_ref[0] + p_ref[1]
    msg = a[:, 0:HD1]
    den = jnp.dot(a[:, HD1:HD1 + 8], em_ref[...],
                  preferred_element_type=jnp.float32,
                  precision=lax.Precision.HIGHEST) + 1e-9
    h = msg / den + b1_ref[...]
    h = jnp.where(h > 0, h, jnp.exp(jnp.minimum(h, 0.0)) - 1.0)
    f2 = jnp.dot(h, w2_ref[...], preferred_element_type=jnp.float32,
                 precision=lax.Precision.HIGHEST)
    el2 = jnp.dot(f2, al2_ref[...], preferred_element_type=jnp.float32,
                  precision=lax.Precision.HIGHEST)
    er2 = jnp.dot(f2, ar2_ref[...], preferred_element_type=jnp.float32,
                  precision=lax.Precision.HIGHEST)
    ones = jnp.ones((1, HD2), jnp.float32)
    t2_ref[...] = jnp.concatenate([f2, el2 * ones], axis=1)
    er2_ref[...] = er2 * ones


def _tc_post(p_ref, b2_ref, o_ref):
    a = p_ref[0] + p_ref[1]
    o_ref[...] = a[:, 0:HD2] / (a[:, HD2:HD2 + 1] + 1e-9) + b2_ref[...]


def _make_sc_body(row_w, first_layer):
    def body(tab_hbm, er_hbm, src_hbm, dst_hbm, out_hbm,
             src_v, dst_v, r0, e0, m0, r1, e1, m1, zbuf, acc,
             g0, g1, s0, s1):
        R = (r0, r1)
        ERc = (e0, e1)
        M = (m0, m1)
        G = (g0, g1)
        S = (s0, s1)
        c = lax.axis_index("c")
        sid = lax.axis_index("s")
        wid = sid * 2 + c

        def _zrow(i, _):
            zv = jnp.zeros((16,), jnp.float32)
            for q in range(row_w // 16):
                zbuf[i, pl.ds(q * 16, 16)] = zv
            return 0
        lax.fori_loop(0, ZB, _zrow, 0)
        for k in range(RPT // ZB):
            pltpu.sync_copy(zbuf, acc.at[pl.ds(sid * RPT + k * ZB, ZB)])
        plsc.subcore_barrier()

        pltpu.sync_copy(src_hbm.at[wid], src_v)
        pltpu.sync_copy(dst_hbm.at[wid], dst_v)

        lane = lax.iota(jnp.int32, 16)
        half = lane >> 3

        def gather_start(j, b):
            pltpu.async_copy(tab_hbm.at[src_v.at[j]], R[b], G[b])
            pltpu.async_copy(er_hbm.at[dst_v.at[j]], ERc[b], G[b])

        def gather_wait(j, b):
            pltpu.make_async_copy(tab_hbm.at[src_v.at[j]], R[b], G[b]).wait()
            pltpu.make_async_copy(er_hbm.at[dst_v.at[j]], ERc[b], G[b]).wait()

        def scatter_start(j, b):
            pltpu.async_copy(M[b], acc.at[dst_v.at[j]], S[b], add=True)

        def scatter_wait(j, b):
            pltpu.make_async_copy(M[b], acc.at[dst_v.at[j]], S[b]).wait()

        def compute(b):
            rows = R[b]
            erc = ERc[b]
            msg = M[b]
            if first_layer:
                def _edge(e, _):
                    elv = rows[e, pl.ds(HD1, 16)]
                    t = elv + erc[e, pl.ds(0, 16)]
                    t = jnp.maximum(t, 0.2 * t)
                    ee = jnp.exp(t)
                    for q in range(4):
                        mq = lax.gather(
                            ee, (half + 2 * q)[:, None],
                            lax.GatherDimensionNumbers(
                                offset_dims=(), collapsed_slice_dims=(0,),
                                start_index_map=(0,)),
                            (1,), mode=lax.GatherScatterMode.PROMISE_IN_BOUNDS)
                        msg[e, pl.ds(q * 16, 16)] = (
                            rows[e, pl.ds(q * 16, 16)] * mq)
                    msg[e, pl.ds(HD1, 16)] = jnp.where(lane < 8, ee, 0.0)
                    return 0
            else:
                def _edge(e, _):
                    t = rows[e, pl.ds(HD2, 16)] + erc[e, pl.ds(0, 16)]
                    t = jnp.maximum(t, 0.2 * t)
                    ee = jnp.exp(t)
                    msg[e, pl.ds(0, 16)] = rows[e, pl.ds(0, 16)] * ee
                    msg[e, pl.ds(HD2, 16)] = jnp.where(lane < 1, ee, 0.0)
                    return 0
            lax.fori_loop(0, CH, _edge, 0)

        gather_start(0, 0)

        def outer(jj, _):
            for b in (0, 1):
                j = 2 * jj + b
                gather_wait(j, b)
                gather_start(jnp.minimum(j + 1, NCH - 1), 1 - b)

                @pl.when(j >= 2)
                def _():
                    scatter_wait(j - 2, b)

                compute(b)
                scatter_start(j, b)
            return 0
        lax.fori_loop(0, NCH // 2, outer, 0)

        scatter_wait(NCH - 2, 0)
        scatter_wait(NCH - 1, 1)
        gather_wait(NCH - 1, 0)

        plsc.subcore_barrier()
        pltpu.sync_copy(acc.at[pl.ds(sid * RPT, RPT)],
                        out_hbm.at[c, pl.ds(sid * RPT, RPT)])

    return body


def _sc_call(row_w, first_layer):
    mesh = plsc.VectorSubcoreMesh(core_axis_name="c", subcore_axis_name="s")
    return pl.kernel(
        _make_sc_body(row_w, first_layer),
        out_type=jax.ShapeDtypeStruct((2, NP, row_w), jnp.float32),
        mesh=mesh,
        scratch_types=[
            pltpu.VMEM((NCH, CH), jnp.int32),
            pltpu.VMEM((NCH, CH), jnp.int32),
            pltpu.VMEM((CH, row_w), jnp.float32),
            pltpu.VMEM((CH, 16), jnp.float32),
            pltpu.VMEM((CH, row_w), jnp.float32),
            pltpu.VMEM((CH, row_w), jnp.float32),
            pltpu.VMEM((CH, 16), jnp.float32),
            pltpu.VMEM((CH, row_w), jnp.float32),
            pltpu.VMEM((ZB, row_w), jnp.float32),
            pltpu.VMEM_SHARED((NP, row_w), jnp.float32),
            pltpu.SemaphoreType.DMA,
            pltpu.SemaphoreType.DMA,
            pltpu.SemaphoreType.DMA,
            pltpu.SemaphoreType.DMA,
        ],
        compiler_params=pltpu.CompilerParams(use_tc_tiling_on_sc=False),
    )


def kernel(x, edge_index, W1, attn_l1, attn_r1, b1, W2, attn_l2, attn_r2, b2):
    f32 = jnp.float32
    src = edge_index[0].astype(jnp.int32).reshape(NW, NCH, CH)
    dst = edge_index[1].astype(jnp.int32).reshape(NW, NCH, CH)

    eye8 = jnp.eye(8, dtype=f32)
    AL = (eye8[:, None, :] * attn_l1.astype(f32)[:, :, None]).reshape(HD1, 8)
    AR = (eye8[:, None, :] * attn_r1.astype(f32)[:, :, None]).reshape(HD1, 8)
    Emat = jnp.repeat(eye8, 8, axis=1)

    t1, er1 = pl.pallas_call(
        _tc_pre1,
        grid=(_GRID,),
        in_specs=[
            pl.BlockSpec((_BLK, IN), lambda i: (i, 0)),
            pl.BlockSpec((IN, HD1), lambda i: (0, 0)),
            pl.BlockSpec((HD1, 8), lambda i: (0, 0)),
            pl.BlockSpec((HD1, 8), lambda i: (0, 0)),
        ],
        out_specs=[
            pl.BlockSpec((_BLK, W1ROW), lambda i: (i, 0)),
            pl.BlockSpec((_BLK, 16), lambda i: (i, 0)),
        ],
        out_shape=[
            jax.ShapeDtypeStruct((N, W1ROW), f32),
            jax.ShapeDtypeStruct((N, 16), f32),
        ],
    )(x.astype(f32), W1.astype(f32), AL, AR)

    parts1 = _sc_call(W1ROW, True)(t1, er1, src, dst)

    t2, er2 = pl.pallas_call(
        _tc_mid,
        grid=(_GRID,),
        in_specs=[
            pl.BlockSpec((2, _BLK, W1ROW), lambda i: (0, i, 0)),
            pl.BlockSpec((1, HD1), lambda i: (0, 0)),
            pl.BlockSpec((8, HD1), lambda i: (0, 0)),
            pl.BlockSpec((HD1, HD2), lambda i: (0, 0)),
            pl.BlockSpec((HD2, 1), lambda i: (0, 0)),
            pl.BlockSpec((HD2, 1), lambda i: (0, 0)),
        ],
        out_specs=[
            pl.BlockSpec((_BLK, W2ROW), lambda i: (i, 0)),
            pl.BlockSpec((_BLK, 16), lambda i: (i, 0)),
        ],
        out_shape=[
            jax.ShapeDtypeStruct((N, W2ROW), f32),
            jax.ShapeDtypeStruct((N, 16), f32),
        ],
    )(parts1, b1.astype(f32).reshape(1, HD1), Emat, W2.astype(f32),
      attn_l2.astype(f32).reshape(HD2, 1), attn_r2.astype(f32).reshape(HD2, 1))

    parts2 = _sc_call(W2ROW, False)(t2, er2, src, dst)

    out = pl.pallas_call(
        _tc_post,
        grid=(_GRID,),
        in_specs=[
            pl.BlockSpec((2, _BLK, W2ROW), lambda i: (0, i, 0)),
            pl.BlockSpec((1, HD2), lambda i: (0, 0)),
        ],
        out_specs=pl.BlockSpec((_BLK, HD2), lambda i: (i, 0)),
        out_shape=jax.ShapeDtypeStruct((N, HD2), f32),
    )(parts2, b2.astype(f32).reshape(1, HD2))

    return out


# speedup vs baseline: 127.2581x; 1.2849x over previous
import functools

import jax
import jax.numpy as jnp
from jax import lax
from jax.experimental import pallas as pl
from jax.experimental.pallas import tpu as pltpu
from jax.experimental.pallas import tpu_sc as plsc

N = 10000
E = 320000
IN = 128
HD1 = 64
W1ROW = 80
HD2 = 16
W2ROW = 32

NW = 32
PT = E // NW
CH = 100
NCH = PT // CH
NP = 10240
RPT = NP // 16
ZB = 128

_BLK = 1000
_GRID = N // _BLK


def _tc_pre1(x_ref, w1_ref, al_ref, ar_ref, t1_ref, er_ref):
    feat = jnp.dot(x_ref[...], w1_ref[...],
                   preferred_element_type=jnp.float32,
                   precision=lax.Precision.HIGHEST)
    el = jnp.dot(feat, al_ref[...], preferred_element_type=jnp.float32,
                 precision=lax.Precision.HIGHEST)
    er = jnp.dot(feat, ar_ref[...], preferred_element_type=jnp.float32,
                 precision=lax.Precision.HIGHEST)
    z8 = jnp.zeros((_BLK, 8), jnp.float32)
    t1_ref[...] = jnp.concatenate([feat, el, z8], axis=1)
    er_ref[...] = jnp.concatenate([er, z8], axis=1)


def _tc_mid(p_ref, b1_ref, em_ref, w2_ref, al2_ref, ar2_ref, t2_ref, er2_ref):
    a = p_ref[0] + p_ref[1]
    msg = a[:, 0:HD1]
    den = jnp.dot(a[:, HD1:HD1 + 8], em_ref[...],
                  preferred_element_type=jnp.float32,
                  precision=lax.Precision.HIGHEST) + 1e-9
    h = msg / den + b1_ref[...]
    h = jnp.where(h > 0, h, jnp.exp(jnp.minimum(h, 0.0)) - 1.0)
    f2 = jnp.dot(h, w2_ref[...], preferred_element_type=jnp.float32,
                 precision=lax.Precision.HIGHEST)
    el2 = jnp.dot(f2, al2_ref[...], preferred_element_type=jnp.float32,
                  precision=lax.Precision.HIGHEST)
    er2 = jnp.dot(f2, ar2_ref[...], preferred_element_type=jnp.float32,
                  precision=lax.Precision.HIGHEST)
    ones = jnp.ones((1, HD2), jnp.float32)
    t2_ref[...] = jnp.concatenate([f2, el2 * ones], axis=1)
    er2_ref[...] = er2 * ones


def _tc_post(p_ref, b2_ref, o_ref):
    a = p_ref[0] + p_ref[1]
    o_ref[...] = a[:, 0:HD2] / (a[:, HD2:HD2 + 1] + 1e-9) + b2_ref[...]


def _make_sc_body(row_w, first_layer):
    def body(tab_hbm, er_hbm, src_hbm, dst_hbm, out_hbm,
             src_v, dst_v, r0, e0, m0, r1, e1, m1, zbuf, acc,
             g0, g1, s0, s1):
        R = (r0, r1)
        ERc = (e0, e1)
        M = (m0, m1)
        G = (g0, g1)
        S = (s0, s1)
        c = lax.axis_index("c")
        sid = lax.axis_index("s")
        wid = sid * 2 + c

        def _zrow(i, _):
            zv = jnp.zeros((16,), jnp.float32)
            for q in range(row_w // 16):
                zbuf[i, pl.ds(q * 16, 16)] = zv
            return 0
        lax.fori_loop(0, ZB, _zrow, 0)
        for k in range(RPT // ZB):
            pltpu.sync_copy(zbuf, acc.at[pl.ds(sid * RPT + k * ZB, ZB)])
        plsc.subcore_barrier()

        pltpu.sync_copy(src_hbm.at[wid], src_v)
        pltpu.sync_copy(dst_hbm.at[wid], dst_v)

        lane = lax.iota(jnp.int32, 16)
        half = lane >> 3

        def gather_start(j, b):
            pltpu.async_copy(tab_hbm.at[src_v.at[j]], R[b], G[b])
            pltpu.async_copy(er_hbm.at[dst_v.at[j]], ERc[b], G[b])

        def gather_wait(j, b):
            pltpu.make_async_copy(tab_hbm.at[src_v.at[j]], R[b], G[b]).wait()
            pltpu.make_async_copy(er_hbm.at[dst_v.at[j]], ERc[b], G[b]).wait()

        def scatter_start(j, b):
            pltpu.async_copy(M[b], acc.at[dst_v.at[j]], S[b], add=True)

        def scatter_wait(j, b):
            pltpu.make_async_copy(M[b], acc.at[dst_v.at[j]], S[b]).wait()

        def compute(b):
            rows = R[b]
            erc = ERc[b]
            msg = M[b]
            if first_layer:
                @plsc.parallel_loop(0, CH, unroll=4)
                def _edge(e):
                    elv = rows[e, pl.ds(HD1, 16)]
                    t = elv + erc[e, pl.ds(0, 16)]
                    t = jnp.maximum(t, 0.2 * t)
                    ee = jnp.exp(t)
                    for q in range(4):
                        mq = lax.gather(
                            ee, (half + 2 * q)[:, None],
                            lax.GatherDimensionNumbers(
                                offset_dims=(), collapsed_slice_dims=(0,),
                                start_index_map=(0,)),
                            (1,), mode=lax.GatherScatterMode.PROMISE_IN_BOUNDS)
                        msg[e, pl.ds(q * 16, 16)] = (
                            rows[e, pl.ds(q * 16, 16)] * mq)
                    msg[e, pl.ds(HD1, 16)] = jnp.where(lane < 8, ee, 0.0)
            else:
                @plsc.parallel_loop(0, CH, unroll=4)
                def _edge(e):
                    t = rows[e, pl.ds(HD2, 16)] + erc[e, pl.ds(0, 16)]
                    t = jnp.maximum(t, 0.2 * t)
                    ee = jnp.exp(t)
                    msg[e, pl.ds(0, 16)] = rows[e, pl.ds(0, 16)] * ee
                    msg[e, pl.ds(HD2, 16)] = jnp.where(lane < 1, ee, 0.0)

        gather_start(0, 0)

        def outer(jj, _):
            for b in (0, 1):
                j = 2 * jj + b
                gather_wait(j, b)
                gather_start(jnp.minimum(j + 1, NCH - 1), 1 - b)

                @pl.when(j >= 2)
                def _():
                    scatter_wait(j - 2, b)

                compute(b)
                scatter_start(j, b)
            return 0
        lax.fori_loop(0, NCH // 2, outer, 0)

        scatter_wait(NCH - 2, 0)
        scatter_wait(NCH - 1, 1)
        gather_wait(NCH - 1, 0)

        plsc.subcore_barrier()
        pltpu.sync_copy(acc.at[pl.ds(sid * RPT, RPT)],
                        out_hbm.at[c, pl.ds(sid * RPT, RPT)])

    return body


def _sc_call(row_w, first_layer):
    mesh = plsc.VectorSubcoreMesh(core_axis_name="c", subcore_axis_name="s")
    return pl.kernel(
        _make_sc_body(row_w, first_layer),
        out_type=jax.ShapeDtypeStruct((2, NP, row_w), jnp.float32),
        mesh=mesh,
        scratch_types=[
            pltpu.VMEM((NCH, CH), jnp.int32),
            pltpu.VMEM((NCH, CH), jnp.int32),
            pltpu.VMEM((CH, row_w), jnp.float32),
            pltpu.VMEM((CH, 16), jnp.float32),
            pltpu.VMEM((CH, row_w), jnp.float32),
            pltpu.VMEM((CH, row_w), jnp.float32),
            pltpu.VMEM((CH, 16), jnp.float32),
            pltpu.VMEM((CH, row_w), jnp.float32),
            pltpu.VMEM((ZB, row_w), jnp.float32),
            pltpu.VMEM_SHARED((NP, row_w), jnp.float32),
            pltpu.SemaphoreType.DMA,
            pltpu.SemaphoreType.DMA,
            pltpu.SemaphoreType.DMA,
            pltpu.SemaphoreType.DMA,
        ],
        compiler_params=pltpu.CompilerParams(use_tc_tiling_on_sc=False),
    )


def kernel(x, edge_index, W1, attn_l1, attn_r1, b1, W2, attn_l2, attn_r2, b2):
    f32 = jnp.float32
    src = edge_index[0].astype(jnp.int32).reshape(NW, NCH, CH)
    dst = edge_index[1].astype(jnp.int32).reshape(NW, NCH, CH)

    eye8 = jnp.eye(8, dtype=f32)
    AL = (eye8[:, None, :] * attn_l1.astype(f32)[:, :, None]).reshape(HD1, 8)
    AR = (eye8[:, None, :] * attn_r1.astype(f32)[:, :, None]).reshape(HD1, 8)
    Emat = jnp.repeat(eye8, 8, axis=1)

    t1, er1 = pl.pallas_call(
        _tc_pre1,
        grid=(_GRID,),
        in_specs=[
            pl.BlockSpec((_BLK, IN), lambda i: (i, 0)),
            pl.BlockSpec((IN, HD1), lambda i: (0, 0)),
            pl.BlockSpec((HD1, 8), lambda i: (0, 0)),
            pl.BlockSpec((HD1, 8), lambda i: (0, 0)),
        ],
        out_specs=[
            pl.BlockSpec((_BLK, W1ROW), lambda i: (i, 0)),
            pl.BlockSpec((_BLK, 16), lambda i: (i, 0)),
        ],
        out_shape=[
            jax.ShapeDtypeStruct((N, W1ROW), f32),
            jax.ShapeDtypeStruct((N, 16), f32),
        ],
    )(x.astype(f32), W1.astype(f32), AL, AR)

    parts1 = _sc_call(W1ROW, True)(t1, er1, src, dst)

    t2, er2 = pl.pallas_call(
        _tc_mid,
        grid=(_GRID,),
        in_specs=[
            pl.BlockSpec((2, _BLK, W1ROW), lambda i: (0, i, 0)),
            pl.BlockSpec((1, HD1), lambda i: (0, 0)),
            pl.BlockSpec((8, HD1), lambda i: (0, 0)),
            pl.BlockSpec((HD1, HD2), lambda i: (0, 0)),
            pl.BlockSpec((HD2, 1), lambda i: (0, 0)),
            pl.BlockSpec((HD2, 1), lambda i: (0, 0)),
        ],
        out_specs=[
            pl.BlockSpec((_BLK, W2ROW), lambda i: (i, 0)),
            pl.BlockSpec((_BLK, 16), lambda i: (i, 0)),
        ],
        out_shape=[
            jax.ShapeDtypeStruct((N, W2ROW), f32),
            jax.ShapeDtypeStruct((N, 16), f32),
        ],
    )(parts1, b1.astype(f32).reshape(1, HD1), Emat, W2.astype(f32),
      attn_l2.astype(f32).reshape(HD2, 1), attn_r2.astype(f32).reshape(HD2, 1))

    parts2 = _sc_call(W2ROW, False)(t2, er2, src, dst)

    out = pl.pallas_call(
        _tc_post,
        grid=(_GRID,),
        in_specs=[
            pl.BlockSpec((2, _BLK, W2ROW), lambda i: (0, i, 0)),
            pl.BlockSpec((1, HD2), lambda i: (0, 0)),
        ],
        out_specs=pl.BlockSpec((_BLK, HD2), lambda i: (i, 0)),
        out_shape=jax.ShapeDtypeStruct((N, HD2), f32),
    )(parts2, b2.astype(f32).reshape(1, HD2))

    return out


# speedup vs baseline: 135.0329x; 1.0611x over previous
import functools

import jax
import jax.numpy as jnp
from jax import lax
from jax.experimental import pallas as pl
from jax.experimental.pallas import tpu as pltpu
from jax.experimental.pallas import tpu_sc as plsc

N = 10000
E = 320000
IN = 128
HD1 = 64
W1ROW = 80
HD2 = 16
W2ROW = 32

NW = 32
PT = E // NW
CH = 125
NCH = PT // CH
NP = 10240
RPT = NP // 16
ZB = 128

_BLK = 1000
_GRID = N // _BLK


def _tc_pre1(x_ref, w1_ref, al_ref, ar_ref, t1_ref, er_ref):
    feat = jnp.dot(x_ref[...], w1_ref[...],
                   preferred_element_type=jnp.float32,
                   precision=lax.Precision.HIGHEST)
    el = jnp.dot(feat, al_ref[...], preferred_element_type=jnp.float32,
                 precision=lax.Precision.HIGHEST)
    er = jnp.dot(feat, ar_ref[...], preferred_element_type=jnp.float32,
                 precision=lax.Precision.HIGHEST)
    z8 = jnp.zeros((_BLK, 8), jnp.float32)
    t1_ref[...] = jnp.concatenate([feat, el, z8], axis=1)
    er_ref[...] = jnp.concatenate([er, z8], axis=1)


def _tc_mid(p_ref, b1_ref, em_ref, w2_ref, al2_ref, ar2_ref, t2_ref, er2_ref):
    a = p_ref[0] + p_ref[1]
    msg = a[:, 0:HD1]
    den = jnp.dot(a[:, HD1:HD1 + 8], em_ref[...],
                  preferred_element_type=jnp.float32,
                  precision=lax.Precision.HIGHEST) + 1e-9
    h = msg / den + b1_ref[...]
    h = jnp.where(h > 0, h, jnp.exp(jnp.minimum(h, 0.0)) - 1.0)
    f2 = jnp.dot(h, w2_ref[...], preferred_element_type=jnp.float32,
                 precision=lax.Precision.HIGHEST)
    el2 = jnp.dot(f2, al2_ref[...], preferred_element_type=jnp.float32,
                  precision=lax.Precision.HIGHEST)
    er2 = jnp.dot(f2, ar2_ref[...], preferred_element_type=jnp.float32,
                  precision=lax.Precision.HIGHEST)
    ones = jnp.ones((1, HD2), jnp.float32)
    t2_ref[...] = jnp.concatenate([f2, el2 * ones], axis=1)
    er2_ref[...] = er2 * ones


def _tc_post(p_ref, b2_ref, o_ref):
    a = p_ref[0] + p_ref[1]
    o_ref[...] = a[:, 0:HD2] / (a[:, HD2:HD2 + 1] + 1e-9) + b2_ref[...]


def _make_sc_body(row_w, first_layer):
    def body(tab_hbm, er_hbm, src_hbm, dst_hbm, out_hbm,
             src_v, dst_v, r0, e0, m0, r1, e1, m1, zbuf, acc,
             g0, g1, s0, s1):
        R = (r0, r1)
        ERc = (e0, e1)
        M = (m0, m1)
        G = (g0, g1)
        S = (s0, s1)
        c = lax.axis_index("c")
        sid = lax.axis_index("s")
        wid = sid * 2 + c

        def _zrow(i, _):
            zv = jnp.zeros((16,), jnp.float32)
            for q in range(row_w // 16):
                zbuf[i, pl.ds(q * 16, 16)] = zv
            return 0
        lax.fori_loop(0, ZB, _zrow, 0)
        for k in range(RPT // ZB):
            pltpu.sync_copy(zbuf, acc.at[pl.ds(sid * RPT + k * ZB, ZB)])
        plsc.subcore_barrier()

        pltpu.sync_copy(src_hbm.at[wid], src_v)
        pltpu.sync_copy(dst_hbm.at[wid], dst_v)

        lane = lax.iota(jnp.int32, 16)
        half = lane >> 3

        def gather_start(j, b):
            pltpu.async_copy(tab_hbm.at[src_v.at[j]], R[b], G[b])
            pltpu.async_copy(er_hbm.at[dst_v.at[j]], ERc[b], G[b])

        def gather_wait(j, b):
            pltpu.make_async_copy(tab_hbm.at[src_v.at[j]], R[b], G[b]).wait()
            pltpu.make_async_copy(er_hbm.at[dst_v.at[j]], ERc[b], G[b]).wait()

        def scatter_start(j, b):
            pltpu.async_copy(M[b], acc.at[dst_v.at[j]], S[b], add=True)

        def scatter_wait(j, b):
            pltpu.make_async_copy(M[b], acc.at[dst_v.at[j]], S[b]).wait()

        def compute(b):
            rows = R[b]
            erc = ERc[b]
            msg = M[b]
            if first_layer:
                @plsc.parallel_loop(0, CH, unroll=8)
                def _edge(e):
                    elv = rows[e, pl.ds(HD1, 16)]
                    t = elv + erc[e, pl.ds(0, 16)]
                    t = jnp.maximum(t, 0.2 * t)
                    ee = jnp.exp(t)
                    for q in range(4):
                        mq = lax.gather(
                            ee, (half + 2 * q)[:, None],
                            lax.GatherDimensionNumbers(
                                offset_dims=(), collapsed_slice_dims=(0,),
                                start_index_map=(0,)),
                            (1,), mode=lax.GatherScatterMode.PROMISE_IN_BOUNDS)
                        msg[e, pl.ds(q * 16, 16)] = (
                            rows[e, pl.ds(q * 16, 16)] * mq)
                    msg[e, pl.ds(HD1, 16)] = jnp.where(lane < 8, ee, 0.0)
            else:
                @plsc.parallel_loop(0, CH, unroll=8)
                def _edge(e):
                    t = rows[e, pl.ds(HD2, 16)] + erc[e, pl.ds(0, 16)]
                    t = jnp.maximum(t, 0.2 * t)
                    ee = jnp.exp(t)
                    msg[e, pl.ds(0, 16)] = rows[e, pl.ds(0, 16)] * ee
                    msg[e, pl.ds(HD2, 16)] = jnp.where(lane < 1, ee, 0.0)

        gather_start(0, 0)

        def outer(jj, _):
            for b in (0, 1):
                j = 2 * jj + b
                gather_wait(j, b)
                gather_start(jnp.minimum(j + 1, NCH - 1), 1 - b)

                @pl.when(j >= 2)
                def _():
                    scatter_wait(j - 2, b)

                compute(b)
                scatter_start(j, b)
            return 0
        lax.fori_loop(0, NCH // 2, outer, 0)

        scatter_wait(NCH - 2, 0)
        scatter_wait(NCH - 1, 1)
        gather_wait(NCH - 1, 0)

        plsc.subcore_barrier()
        pltpu.sync_copy(acc.at[pl.ds(sid * RPT, RPT)],
                        out_hbm.at[c, pl.ds(sid * RPT, RPT)])

    return body


def _sc_call(row_w, first_layer):
    mesh = plsc.VectorSubcoreMesh(core_axis_name="c", subcore_axis_name="s")
    return pl.kernel(
        _make_sc_body(row_w, first_layer),
        out_type=jax.ShapeDtypeStruct((2, NP, row_w), jnp.float32),
        mesh=mesh,
        scratch_types=[
            pltpu.VMEM((NCH, CH), jnp.int32),
            pltpu.VMEM((NCH, CH), jnp.int32),
            pltpu.VMEM((CH, row_w), jnp.float32),
            pltpu.VMEM((CH, 16), jnp.float32),
            pltpu.VMEM((CH, row_w), jnp.float32),
            pltpu.VMEM((CH, row_w), jnp.float32),
            pltpu.VMEM((CH, 16), jnp.float32),
            pltpu.VMEM((CH, row_w), jnp.float32),
            pltpu.VMEM((ZB, row_w), jnp.float32),
            pltpu.VMEM_SHARED((NP, row_w), jnp.float32),
            pltpu.SemaphoreType.DMA,
            pltpu.SemaphoreType.DMA,
            pltpu.SemaphoreType.DMA,
            pltpu.SemaphoreType.DMA,
        ],
        compiler_params=pltpu.CompilerParams(use_tc_tiling_on_sc=False),
    )


def kernel(x, edge_index, W1, attn_l1, attn_r1, b1, W2, attn_l2, attn_r2, b2):
    f32 = jnp.float32
    src = edge_index[0].astype(jnp.int32).reshape(NW, NCH, CH)
    dst = edge_index[1].astype(jnp.int32).reshape(NW, NCH, CH)

    eye8 = jnp.eye(8, dtype=f32)
    AL = (eye8[:, None, :] * attn_l1.astype(f32)[:, :, None]).reshape(HD1, 8)
    AR = (eye8[:, None, :] * attn_r1.astype(f32)[:, :, None]).reshape(HD1, 8)
    Emat = jnp.repeat(eye8, 8, axis=1)

    t1, er1 = pl.pallas_call(
        _tc_pre1,
        grid=(_GRID,),
        in_specs=[
            pl.BlockSpec((_BLK, IN), lambda i: (i, 0)),
            pl.BlockSpec((IN, HD1), lambda i: (0, 0)),
            pl.BlockSpec((HD1, 8), lambda i: (0, 0)),
            pl.BlockSpec((HD1, 8), lambda i: (0, 0)),
        ],
        out_specs=[
            pl.BlockSpec((_BLK, W1ROW), lambda i: (i, 0)),
            pl.BlockSpec((_BLK, 16), lambda i: (i, 0)),
        ],
        out_shape=[
            jax.ShapeDtypeStruct((N, W1ROW), f32),
            jax.ShapeDtypeStruct((N, 16), f32),
        ],
    )(x.astype(f32), W1.astype(f32), AL, AR)

    parts1 = _sc_call(W1ROW, True)(t1, er1, src, dst)

    t2, er2 = pl.pallas_call(
        _tc_mid,
        grid=(_GRID,),
        in_specs=[
            pl.BlockSpec((2, _BLK, W1ROW), lambda i: (0, i, 0)),
            pl.BlockSpec((1, HD1), lambda i: (0, 0)),
            pl.BlockSpec((8, HD1), lambda i: (0, 0)),
            pl.BlockSpec((HD1, HD2), lambda i: (0, 0)),
            pl.BlockSpec((HD2, 1), lambda i: (0, 0)),
            pl.BlockSpec((HD2, 1), lambda i: (0, 0)),
        ],
        out_specs=[
            pl.BlockSpec((_BLK, W2ROW), lambda i: (i, 0)),
            pl.BlockSpec((_BLK, 16), lambda i: (i, 0)),
        ],
        out_shape=[
            jax.ShapeDtypeStruct((N, W2ROW), f32),
            jax.ShapeDtypeStruct((N, 16), f32),
        ],
    )(parts1, b1.astype(f32).reshape(1, HD1), Emat, W2.astype(f32),
      attn_l2.astype(f32).reshape(HD2, 1), attn_r2.astype(f32).reshape(HD2, 1))

    parts2 = _sc_call(W2ROW, False)(t2, er2, src, dst)

    out = pl.pallas_call(
        _tc_post,
        grid=(_GRID,),
        in_specs=[
            pl.BlockSpec((2, _BLK, W2ROW), lambda i: (0, i, 0)),
            pl.BlockSpec((1, HD2), lambda i: (0, 0)),
        ],
        out_specs=pl.BlockSpec((_BLK, HD2), lambda i: (i, 0)),
        out_shape=jax.ShapeDtypeStruct((N, HD2), f32),
    )(parts2, b2.astype(f32).reshape(1, HD2))

    return out


# speedup vs baseline: 163.5010x; 1.2108x over previous
import functools

import jax
import jax.numpy as jnp
from jax import lax
from jax.experimental import pallas as pl
from jax.experimental.pallas import tpu as pltpu
from jax.experimental.pallas import tpu_sc as plsc

N = 10000
E = 320000
IN = 128
HD1 = 64
W1ROW = 80
HD2 = 16
W2ROW = 32

NW = 32
PT = E // NW
CH = 125
NCH = PT // CH
NP = 10240
RPT = NP // 16
ZB = 128

_BLK = 1000
_GRID = N // _BLK


def _tc_pre1(x_ref, w1_ref, alr_ref, t1_ref, er_ref):
    feat = jnp.dot(x_ref[...], w1_ref[...],
                   preferred_element_type=jnp.float32)
    elr = jnp.dot(feat, alr_ref[...],
                  preferred_element_type=jnp.float32)
    z8 = jnp.zeros((_BLK, 8), jnp.float32)
    t1_ref[...] = jnp.concatenate([feat, elr[:, 0:8], z8], axis=1)
    er_ref[...] = jnp.concatenate([elr[:, 8:16], z8], axis=1)


def _tc_mid(p_ref, b1_ref, em_ref, w2_ref, alr2_ref, t2_ref, er2_ref):
    a = p_ref[0] + p_ref[1]
    msg = a[:, 0:HD1]
    den = jnp.dot(a[:, HD1:HD1 + 8], em_ref[...],
                  preferred_element_type=jnp.float32) + 1e-9
    h = msg / den + b1_ref[...]
    h = jnp.where(h > 0, h, jnp.exp(jnp.minimum(h, 0.0)) - 1.0)
    f2 = jnp.dot(h, w2_ref[...], preferred_element_type=jnp.float32)
    elr2 = jnp.dot(f2, alr2_ref[...],
                   preferred_element_type=jnp.float32)
    ones = jnp.ones((1, HD2), jnp.float32)
    t2_ref[...] = jnp.concatenate([f2, elr2[:, 0:1] * ones], axis=1)
    er2_ref[...] = elr2[:, 1:2] * ones


def _tc_post(p_ref, b2_ref, o_ref):
    a = p_ref[0] + p_ref[1]
    o_ref[...] = a[:, 0:HD2] / (a[:, HD2:HD2 + 1] + 1e-9) + b2_ref[...]


def _make_sc_body(row_w, first_layer):
    def body(tab_hbm, er_hbm, src_hbm, dst_hbm, out_hbm,
             src_v, dst_v, r0, e0, m0, r1, e1, m1, zbuf, acc,
             g0, g1, s0, s1):
        R = (r0, r1)
        ERc = (e0, e1)
        M = (m0, m1)
        G = (g0, g1)
        S = (s0, s1)
        c = lax.axis_index("c")
        sid = lax.axis_index("s")
        wid = sid * 2 + c

        def _zrow(i, _):
            zv = jnp.zeros((16,), jnp.float32)
            for q in range(row_w // 16):
                zbuf[i, pl.ds(q * 16, 16)] = zv
            return 0
        lax.fori_loop(0, ZB, _zrow, 0)
        for k in range(RPT // ZB):
            pltpu.sync_copy(zbuf, acc.at[pl.ds(sid * RPT + k * ZB, ZB)])
        plsc.subcore_barrier()

        pltpu.sync_copy(src_hbm.at[wid], src_v)
        pltpu.sync_copy(dst_hbm.at[wid], dst_v)

        lane = lax.iota(jnp.int32, 16)
        half = lane >> 3

        def gather_start(j, b):
            pltpu.async_copy(tab_hbm.at[src_v.at[j]], R[b], G[b])
            pltpu.async_copy(er_hbm.at[dst_v.at[j]], ERc[b], G[b])

        def gather_wait(j, b):
            pltpu.make_async_copy(tab_hbm.at[src_v.at[j]], R[b], G[b]).wait()
            pltpu.make_async_copy(er_hbm.at[dst_v.at[j]], ERc[b], G[b]).wait()

        def scatter_start(j, b):
            pltpu.async_copy(M[b], acc.at[dst_v.at[j]], S[b], add=True)

        def scatter_wait(j, b):
            pltpu.make_async_copy(M[b], acc.at[dst_v.at[j]], S[b]).wait()

        def compute(b):
            rows = R[b]
            erc = ERc[b]
            msg = M[b]
            if first_layer:
                @plsc.parallel_loop(0, CH, unroll=8)
                def _edge(e):
                    elv = rows[e, pl.ds(HD1, 16)]
                    t = elv + erc[e, pl.ds(0, 16)]
                    t = jnp.maximum(t, 0.2 * t)
                    ee = jnp.exp(t)
                    for q in range(4):
                        mq = lax.gather(
                            ee, (half + 2 * q)[:, None],
                            lax.GatherDimensionNumbers(
                                offset_dims=(), collapsed_slice_dims=(0,),
                                start_index_map=(0,)),
                            (1,), mode=lax.GatherScatterMode.PROMISE_IN_BOUNDS)
                        msg[e, pl.ds(q * 16, 16)] = (
                            rows[e, pl.ds(q * 16, 16)] * mq)
                    msg[e, pl.ds(HD1, 16)] = jnp.where(lane < 8, ee, 0.0)
            else:
                @plsc.parallel_loop(0, CH, unroll=8)
                def _edge(e):
                    t = rows[e, pl.ds(HD2, 16)] + erc[e, pl.ds(0, 16)]
                    t = jnp.maximum(t, 0.2 * t)
                    ee = jnp.exp(t)
                    msg[e, pl.ds(0, 16)] = rows[e, pl.ds(0, 16)] * ee
                    msg[e, pl.ds(HD2, 16)] = jnp.where(lane < 1, ee, 0.0)

        gather_start(0, 0)

        def outer(jj, _):
            for b in (0, 1):
                j = 2 * jj + b
                gather_wait(j, b)
                gather_start(jnp.minimum(j + 1, NCH - 1), 1 - b)

                @pl.when(j >= 2)
                def _():
                    scatter_wait(j - 2, b)

                compute(b)
                scatter_start(j, b)
            return 0
        lax.fori_loop(0, NCH // 2, outer, 0)

        scatter_wait(NCH - 2, 0)
        scatter_wait(NCH - 1, 1)
        gather_wait(NCH - 1, 0)

        plsc.subcore_barrier()
        pltpu.sync_copy(acc.at[pl.ds(sid * RPT, RPT)],
                        out_hbm.at[c, pl.ds(sid * RPT, RPT)])

    return body


def _sc_call(row_w, first_layer):
    mesh = plsc.VectorSubcoreMesh(core_axis_name="c", subcore_axis_name="s")
    return pl.kernel(
        _make_sc_body(row_w, first_layer),
        out_type=jax.ShapeDtypeStruct((2, NP, row_w), jnp.float32),
        mesh=mesh,
        scratch_types=[
            pltpu.VMEM((NCH, CH), jnp.int32),
            pltpu.VMEM((NCH, CH), jnp.int32),
            pltpu.VMEM((CH, row_w), jnp.float32),
            pltpu.VMEM((CH, 16), jnp.float32),
            pltpu.VMEM((CH, row_w), jnp.float32),
            pltpu.VMEM((CH, row_w), jnp.float32),
            pltpu.VMEM((CH, 16), jnp.float32),
            pltpu.VMEM((CH, row_w), jnp.float32),
            pltpu.VMEM((ZB, row_w), jnp.float32),
            pltpu.VMEM_SHARED((NP, row_w), jnp.float32),
            pltpu.SemaphoreType.DMA,
            pltpu.SemaphoreType.DMA,
            pltpu.SemaphoreType.DMA,
            pltpu.SemaphoreType.DMA,
        ],
        compiler_params=pltpu.CompilerParams(use_tc_tiling_on_sc=False),
    )


def kernel(x, edge_index, W1, attn_l1, attn_r1, b1, W2, attn_l2, attn_r2, b2):
    f32 = jnp.float32
    src = edge_index[0].astype(jnp.int32).reshape(NW, NCH, CH)
    dst = edge_index[1].astype(jnp.int32).reshape(NW, NCH, CH)

    eye8 = jnp.eye(8, dtype=f32)
    AL = (eye8[:, None, :] * attn_l1.astype(f32)[:, :, None]).reshape(HD1, 8)
    AR = (eye8[:, None, :] * attn_r1.astype(f32)[:, :, None]).reshape(HD1, 8)
    Emat = jnp.repeat(eye8, 8, axis=1)

    t1, er1 = pl.pallas_call(
        _tc_pre1,
        grid=(_GRID,),
        in_specs=[
            pl.BlockSpec((_BLK, IN), lambda i: (i, 0)),
            pl.BlockSpec((IN, HD1), lambda i: (0, 0)),
            pl.BlockSpec((HD1, 16), lambda i: (0, 0)),
        ],
        out_specs=[
            pl.BlockSpec((_BLK, W1ROW), lambda i: (i, 0)),
            pl.BlockSpec((_BLK, 16), lambda i: (i, 0)),
        ],
        out_shape=[
            jax.ShapeDtypeStruct((N, W1ROW), f32),
            jax.ShapeDtypeStruct((N, 16), f32),
        ],
    )(x.astype(f32), W1.astype(f32), jnp.concatenate([AL, AR], axis=1))

    parts1 = _sc_call(W1ROW, True)(t1, er1, src, dst)

    t2, er2 = pl.pallas_call(
        _tc_mid,
        grid=(_GRID,),
        in_specs=[
            pl.BlockSpec((2, _BLK, W1ROW), lambda i: (0, i, 0)),
            pl.BlockSpec((1, HD1), lambda i: (0, 0)),
            pl.BlockSpec((8, HD1), lambda i: (0, 0)),
            pl.BlockSpec((HD1, HD2), lambda i: (0, 0)),
            pl.BlockSpec((HD2, 2), lambda i: (0, 0)),
        ],
        out_specs=[
            pl.BlockSpec((_BLK, W2ROW), lambda i: (i, 0)),
            pl.BlockSpec((_BLK, 16), lambda i: (i, 0)),
        ],
        out_shape=[
            jax.ShapeDtypeStruct((N, W2ROW), f32),
            jax.ShapeDtypeStruct((N, 16), f32),
        ],
    )(parts1, b1.astype(f32).reshape(1, HD1), Emat, W2.astype(f32),
      jnp.concatenate([attn_l2.astype(f32).reshape(HD2, 1),
                       attn_r2.astype(f32).reshape(HD2, 1)], axis=1))

    parts2 = _sc_call(W2ROW, False)(t2, er2, src, dst)

    out = pl.pallas_call(
        _tc_post,
        grid=(_GRID,),
        in_specs=[
            pl.BlockSpec((2, _BLK, W2ROW), lambda i: (0, i, 0)),
            pl.BlockSpec((1, HD2), lambda i: (0, 0)),
        ],
        out_specs=pl.BlockSpec((_BLK, HD2), lambda i: (i, 0)),
        out_shape=jax.ShapeDtypeStruct((N, HD2), f32),
    )(parts2, b2.astype(f32).reshape(1, HD2))

    return out


# speedup vs baseline: 167.4037x; 1.0239x over previous
import functools

import jax
import jax.numpy as jnp
from jax import lax
from jax.experimental import pallas as pl
from jax.experimental.pallas import tpu as pltpu
from jax.experimental.pallas import tpu_sc as plsc

N = 10000
E = 320000
IN = 128
HD1 = 64
W1ROW = 80
HD2 = 16
W2ROW = 32

NW = 32
PT = E // NW
CH = 125
NCH = PT // CH
NP = 10240
RPT = NP // 16
ZB = 128

_BLK = 2000
_GRID = N // _BLK


def _tc_pre1(x_ref, w1_ref, alr_ref, t1_ref, er_ref):
    feat = jnp.dot(x_ref[...], w1_ref[...],
                   preferred_element_type=jnp.float32)
    elr = jnp.dot(feat, alr_ref[...],
                  preferred_element_type=jnp.float32)
    z8 = jnp.zeros((_BLK, 8), jnp.float32)
    t1_ref[...] = jnp.concatenate([feat, elr[:, 0:8], z8], axis=1)
    er_ref[...] = jnp.concatenate([elr[:, 8:16], z8], axis=1)


def _tc_mid(p_ref, b1_ref, em_ref, w2_ref, alr2_ref, t2_ref, er2_ref):
    a = p_ref[0] + p_ref[1]
    msg = a[:, 0:HD1]
    den = jnp.dot(a[:, HD1:HD1 + 8], em_ref[...],
                  preferred_element_type=jnp.float32) + 1e-9
    h = msg / den + b1_ref[...]
    h = jnp.where(h > 0, h, jnp.exp(jnp.minimum(h, 0.0)) - 1.0)
    f2 = jnp.dot(h, w2_ref[...], preferred_element_type=jnp.float32)
    elr2 = jnp.dot(f2, alr2_ref[...],
                   preferred_element_type=jnp.float32)
    ones = jnp.ones((1, HD2), jnp.float32)
    t2_ref[...] = jnp.concatenate([f2, elr2[:, 0:1] * ones], axis=1)
    er2_ref[...] = elr2[:, 1:2] * ones


def _tc_post(p_ref, b2_ref, o_ref):
    a = p_ref[0] + p_ref[1]
    o_ref[...] = a[:, 0:HD2] / (a[:, HD2:HD2 + 1] + 1e-9) + b2_ref[...]


def _make_sc_body(row_w, first_layer):
    def body(tab_hbm, er_hbm, src_hbm, dst_hbm, out_hbm,
             src_v, dst_v, r0, e0, m0, r1, e1, m1, zbuf, acc,
             g0, g1, s0, s1):
        R = (r0, r1)
        ERc = (e0, e1)
        M = (m0, m1)
        G = (g0, g1)
        S = (s0, s1)
        c = lax.axis_index("c")
        sid = lax.axis_index("s")
        wid = sid * 2 + c

        def _zrow(i, _):
            zv = jnp.zeros((16,), jnp.float32)
            for q in range(row_w // 16):
                zbuf[i, pl.ds(q * 16, 16)] = zv
            return 0
        lax.fori_loop(0, ZB, _zrow, 0)
        for k in range(RPT // ZB):
            pltpu.sync_copy(zbuf, acc.at[pl.ds(sid * RPT + k * ZB, ZB)])
        plsc.subcore_barrier()

        pltpu.sync_copy(src_hbm.at[wid], src_v)
        pltpu.sync_copy(dst_hbm.at[wid], dst_v)

        lane = lax.iota(jnp.int32, 16)
        half = lane >> 3

        def gather_start(j, b):
            pltpu.async_copy(tab_hbm.at[src_v.at[j]], R[b], G[b])
            pltpu.async_copy(er_hbm.at[dst_v.at[j]], ERc[b], G[b])

        def gather_wait(j, b):
            pltpu.make_async_copy(tab_hbm.at[src_v.at[j]], R[b], G[b]).wait()
            pltpu.make_async_copy(er_hbm.at[dst_v.at[j]], ERc[b], G[b]).wait()

        def scatter_start(j, b):
            pltpu.async_copy(M[b], acc.at[dst_v.at[j]], S[b], add=True)

        def scatter_wait(j, b):
            pltpu.make_async_copy(M[b], acc.at[dst_v.at[j]], S[b]).wait()

        def compute(b):
            rows = R[b]
            erc = ERc[b]
            msg = M[b]
            if first_layer:
                @plsc.parallel_loop(0, CH, unroll=8)
                def _edge(e):
                    elv = rows[e, pl.ds(HD1, 16)]
                    t = elv + erc[e, pl.ds(0, 16)]
                    t = jnp.maximum(t, 0.2 * t)
                    ee = jnp.exp(t)
                    for q in range(4):
                        mq = lax.gather(
                            ee, (half + 2 * q)[:, None],
                            lax.GatherDimensionNumbers(
                                offset_dims=(), collapsed_slice_dims=(0,),
                                start_index_map=(0,)),
                            (1,), mode=lax.GatherScatterMode.PROMISE_IN_BOUNDS)
                        msg[e, pl.ds(q * 16, 16)] = (
                            rows[e, pl.ds(q * 16, 16)] * mq)
                    msg[e, pl.ds(HD1, 16)] = jnp.where(lane < 8, ee, 0.0)
            else:
                @plsc.parallel_loop(0, CH, unroll=16)
                def _edge(e):
                    t = rows[e, pl.ds(HD2, 16)] + erc[e, pl.ds(0, 16)]
                    t = jnp.maximum(t, 0.2 * t)
                    ee = jnp.exp(t)
                    msg[e, pl.ds(0, 16)] = rows[e, pl.ds(0, 16)] * ee
                    msg[e, pl.ds(HD2, 16)] = jnp.where(lane < 1, ee, 0.0)

        gather_start(0, 0)

        def outer(jj, _):
            for b in (0, 1):
                j = 2 * jj + b
                gather_wait(j, b)
                gather_start(jnp.minimum(j + 1, NCH - 1), 1 - b)

                @pl.when(j >= 2)
                def _():
                    scatter_wait(j - 2, b)

                compute(b)
                scatter_start(j, b)
            return 0
        lax.fori_loop(0, NCH // 2, outer, 0)

        scatter_wait(NCH - 2, 0)
        scatter_wait(NCH - 1, 1)
        gather_wait(NCH - 1, 0)

        plsc.subcore_barrier()
        pltpu.sync_copy(acc.at[pl.ds(sid * RPT, RPT)],
                        out_hbm.at[c, pl.ds(sid * RPT, RPT)])

    return body


def _sc_call(row_w, first_layer):
    mesh = plsc.VectorSubcoreMesh(core_axis_name="c", subcore_axis_name="s")
    return pl.kernel(
        _make_sc_body(row_w, first_layer),
        out_type=jax.ShapeDtypeStruct((2, NP, row_w), jnp.float32),
        mesh=mesh,
        scratch_types=[
            pltpu.VMEM((NCH, CH), jnp.int32),
            pltpu.VMEM((NCH, CH), jnp.int32),
            pltpu.VMEM((CH, row_w), jnp.float32),
            pltpu.VMEM((CH, 16), jnp.float32),
            pltpu.VMEM((CH, row_w), jnp.float32),
            pltpu.VMEM((CH, row_w), jnp.float32),
            pltpu.VMEM((CH, 16), jnp.float32),
            pltpu.VMEM((CH, row_w), jnp.float32),
            pltpu.VMEM((ZB, row_w), jnp.float32),
            pltpu.VMEM_SHARED((NP, row_w), jnp.float32),
            pltpu.SemaphoreType.DMA,
            pltpu.SemaphoreType.DMA,
            pltpu.SemaphoreType.DMA,
            pltpu.SemaphoreType.DMA,
        ],
        compiler_params=pltpu.CompilerParams(use_tc_tiling_on_sc=False),
    )


def kernel(x, edge_index, W1, attn_l1, attn_r1, b1, W2, attn_l2, attn_r2, b2):
    f32 = jnp.float32
    src = edge_index[0].astype(jnp.int32).reshape(NW, NCH, CH)
    dst = edge_index[1].astype(jnp.int32).reshape(NW, NCH, CH)

    eye8 = jnp.eye(8, dtype=f32)
    AL = (eye8[:, None, :] * attn_l1.astype(f32)[:, :, None]).reshape(HD1, 8)
    AR = (eye8[:, None, :] * attn_r1.astype(f32)[:, :, None]).reshape(HD1, 8)
    Emat = jnp.repeat(eye8, 8, axis=1)

    t1, er1 = pl.pallas_call(
        _tc_pre1,
        grid=(_GRID,),
        in_specs=[
            pl.BlockSpec((_BLK, IN), lambda i: (i, 0)),
            pl.BlockSpec((IN, HD1), lambda i: (0, 0)),
            pl.BlockSpec((HD1, 16), lambda i: (0, 0)),
        ],
        out_specs=[
            pl.BlockSpec((_BLK, W1ROW), lambda i: (i, 0)),
            pl.BlockSpec((_BLK, 16), lambda i: (i, 0)),
        ],
        out_shape=[
            jax.ShapeDtypeStruct((N, W1ROW), f32),
            jax.ShapeDtypeStruct((N, 16), f32),
        ],
    )(x.astype(f32), W1.astype(f32), jnp.concatenate([AL, AR], axis=1))

    parts1 = _sc_call(W1ROW, True)(t1, er1, src, dst)

    t2, er2 = pl.pallas_call(
        _tc_mid,
        grid=(_GRID,),
        in_specs=[
            pl.BlockSpec((2, _BLK, W1ROW), lambda i: (0, i, 0)),
            pl.BlockSpec((1, HD1), lambda i: (0, 0)),
            pl.BlockSpec((8, HD1), lambda i: (0, 0)),
            pl.BlockSpec((HD1, HD2), lambda i: (0, 0)),
            pl.BlockSpec((HD2, 2), lambda i: (0, 0)),
        ],
        out_specs=[
            pl.BlockSpec((_BLK, W2ROW), lambda i: (i, 0)),
            pl.BlockSpec((_BLK, 16), lambda i: (i, 0)),
        ],
        out_shape=[
            jax.ShapeDtypeStruct((N, W2ROW), f32),
            jax.ShapeDtypeStruct((N, 16), f32),
        ],
    )(parts1, b1.astype(f32).reshape(1, HD1), Emat, W2.astype(f32),
      jnp.concatenate([attn_l2.astype(f32).reshape(HD2, 1),
                       attn_r2.astype(f32).reshape(HD2, 1)], axis=1))

    parts2 = _sc_call(W2ROW, False)(t2, er2, src, dst)

    out = pl.pallas_call(
        _tc_post,
        grid=(_GRID,),
        in_specs=[
            pl.BlockSpec((2, _BLK, W2ROW), lambda i: (0, i, 0)),
            pl.BlockSpec((1, HD2), lambda i: (0, 0)),
        ],
        out_specs=pl.BlockSpec((_BLK, HD2), lambda i: (i, 0)),
        out_shape=jax.ShapeDtypeStruct((N, HD2), f32),
    )(parts2, b2.astype(f32).reshape(1, HD2))

    return out
